# baseline (speedup 1.0000x reference)
import jax
import jax.numpy as jnp
from jax import lax
from jax.experimental import pallas as pl
from jax.experimental.pallas import tpu as pltpu
from jax.experimental.pallas import tpu_sc as plsc

B, L, D = 4096, 200, 128
TROWS = L + 8

_info = plsc.get_sparse_core_info()
NC, NS, LANES = _info.num_cores, _info.num_subcores, _info.num_lanes
NW = NC * NS
ROWS = B * L
ROWS_PER_W = ROWS // NW
CHUNK = 128
STEPS = ROWS_PER_W // CHUNK
NSLOT = 4
VPC = CHUNK // LANES

_mesh = plsc.VectorSubcoreMesh(core_axis_name="c", subcore_axis_name="s")


def _wrap(p):
    return jnp.where(p >= L, p - L, p)


def _pos_encoding_sc(mask_hbm, table_hbm, out_hbm, mask_v,
                     idx_s, rows_v, tab_sh, semg, semw):
    sid = lax.axis_index("s")
    wid = sid * NC + lax.axis_index("c")
    base_w = wid * ROWS_PER_W
    tab_base = sid * TROWS
    lane = lax.iota(jnp.int32, LANES)

    pltpu.async_copy(table_hbm.at[pl.ds(wid * TROWS, TROWS)],
                     tab_sh.at[pl.ds(tab_base, TROWS)], semw[0])
    pltpu.async_copy(mask_hbm.at[pl.ds(base_w, ROWS_PER_W)], mask_v,
                     semw[1])
    pltpu.make_async_copy(table_hbm.at[pl.ds(wid * TROWS, TROWS)],
                          tab_sh.at[pl.ds(tab_base, TROWS)], semw[0]).wait()
    pltpu.make_async_copy(mask_hbm.at[pl.ds(base_w, ROWS_PER_W)], mask_v,
                          semw[1]).wait()

    def gather(c, slot):
        return pltpu.async_copy(tab_sh.at[idx_s[slot]],
                                rows_v.at[slot], semg[slot])

    def write(c, slot):
        return pltpu.async_copy(rows_v.at[slot],
                                out_hbm.at[pl.ds(base_w + c * CHUNK, CHUNK)],
                                semw[slot])

    pos = lane
    for c0 in (0, 1):
        p = pos
        for v in range(VPC):
            m = mask_v[pl.ds(c0 * CHUNK + v * LANES, LANES)]
            zrow = tab_base + L + (v % 8)
            idx_s[c0][pl.ds(v * LANES, LANES)] = jnp.where(
                m != 0, zrow, tab_base + p)
            p = _wrap(p + LANES)
        gather(c0, c0)
        pos = p

    def body2(k, pos):
        for b in range(NSLOT):
            c = 4 * k + b
            sg = (b + 2) % NSLOT

            @pl.when(c >= 2)
            def _():
                pltpu.make_async_copy(
                    rows_v.at[sg],
                    out_hbm.at[pl.ds(base_w, CHUNK)], semw[sg]).wait()

            @pl.when(c + 2 < STEPS)
            def _():
                p = pos
                for v in range(VPC):
                    m = mask_v[pl.ds((c + 2) * CHUNK + v * LANES, LANES)]
                    zrow = tab_base + L + (v % 8)
                    idx_s[sg][pl.ds(v * LANES, LANES)] = jnp.where(
                        m != 0, zrow, tab_base + p)
                    p = _wrap(p + LANES)
                gather(c + 2, sg)

            pltpu.make_async_copy(
                table_hbm.at[idx_s[b]], rows_v.at[b], semg[b]).wait()
            write(c, b)
            pos = _wrap(pos + CHUNK)
        return pos

    lax.fori_loop(0, STEPS // NSLOT, body2, pos)

    for b in (2, 3):
        pltpu.make_async_copy(
            rows_v.at[b], out_hbm.at[pl.ds(base_w, CHUNK)], semw[b]).wait()


_sc_call = pl.kernel(
    _pos_encoding_sc,
    mesh=_mesh,
    out_type=jax.ShapeDtypeStruct((ROWS, D), jnp.float32),
    scratch_types=[
        pltpu.VMEM((ROWS_PER_W,), jnp.int32),
        [pltpu.VMEM((CHUNK,), jnp.int32)] * NSLOT,
        pltpu.VMEM((NSLOT, CHUNK, D), jnp.float32),
        pltpu.VMEM_SHARED((NS * TROWS, D), jnp.float32),
        [pltpu.SemaphoreType.DMA] * NSLOT,
        [pltpu.SemaphoreType.DMA] * NSLOT,
    ],
)


def kernel(x_shape, padding_mask, sinusoid_table):
    mask_flat = padding_mask.reshape(-1).astype(jnp.int32)
    table_ext = jnp.concatenate(
        [sinusoid_table, jnp.zeros((TROWS - L, D), jnp.float32)], axis=0)
    table_rep = jnp.tile(table_ext, (NW, 1))
    out = _sc_call(mask_flat, table_rep)
    return out.reshape(B, L, D)
